# baseline (speedup 1.0000x reference)
import jax
import jax.numpy as jnp
from jax.experimental import pallas as pl


_TM = 1024


def _chamfer_kernel(x_ref, y_ref, dx_ref, dy_ref):
    m = pl.program_id(1)
    xb = x_ref[0]
    yb = y_ref[0]
    x2 = jnp.sum(xb * xb, axis=0)
    y2 = jnp.sum(yb * yb, axis=0)
    xy = jax.lax.dot_general(
        xb, yb, (((0,), (0,)), ((), ())),
        preferred_element_type=jnp.float32)
    d = jnp.maximum(x2[:, None] + y2[None, :] - 2.0 * xy, 0.0)
    dy_ref[0, 0] = jnp.min(d, axis=0)
    part = jnp.min(d, axis=1)

    @pl.when(m == 0)
    def _():
        dx_ref[0, 0] = part

    @pl.when(m > 0)
    def _():
        dx_ref[0, 0] = jnp.minimum(dx_ref[0, 0], part)


def kernel(x, y):
    B, C, N = x.shape
    M = y.shape[2]
    dx, dy = pl.pallas_call(
        _chamfer_kernel,
        grid=(B, M // _TM),
        in_specs=[
            pl.BlockSpec((1, C, N), lambda b, m: (b, 0, 0)),
            pl.BlockSpec((1, C, _TM), lambda b, m: (b, 0, m)),
        ],
        out_specs=[
            pl.BlockSpec((1, 1, N), lambda b, m: (b, 0, 0)),
            pl.BlockSpec((1, 1, _TM), lambda b, m: (b, 0, m)),
        ],
        out_shape=[
            jax.ShapeDtypeStruct((B, 1, N), jnp.float32),
            jax.ShapeDtypeStruct((B, 1, M), jnp.float32),
        ],
    )(x, y)
    return jnp.mean(dx) + jnp.mean(dy) * 10.0


# speedup vs baseline: 1.1789x; 1.1789x over previous
import jax
import jax.numpy as jnp
from jax.experimental import pallas as pl


_TM = 1024


def _chamfer_kernel(x_ref, y_ref, dx_ref, dy_ref):
    m = pl.program_id(1)
    xb = x_ref[0]
    yb = y_ref[0]
    x2 = jnp.sum(xb * xb, axis=0)
    y2 = jnp.sum(yb * yb, axis=0)
    xy2 = jax.lax.dot_general(
        xb, yb * -2.0, (((0,), (0,)), ((), ())),
        preferred_element_type=jnp.float32)
    d = (xy2 + x2[:, None]) + y2[None, :]
    dy_ref[0, 0] = jnp.maximum(jnp.min(d, axis=0), 0.0)
    part = jnp.min(d, axis=1)

    @pl.when(m == 0)
    def _():
        dx_ref[0, 0] = part

    @pl.when(m > 0)
    def _():
        dx_ref[0, 0] = jnp.minimum(dx_ref[0, 0], part)

    @pl.when(m == pl.num_programs(1) - 1)
    def _():
        dx_ref[0, 0] = jnp.maximum(dx_ref[0, 0], 0.0)


def kernel(x, y):
    B, C, N = x.shape
    M = y.shape[2]
    dx, dy = pl.pallas_call(
        _chamfer_kernel,
        grid=(B, M // _TM),
        in_specs=[
            pl.BlockSpec((1, C, N), lambda b, m: (b, 0, 0)),
            pl.BlockSpec((1, C, _TM), lambda b, m: (b, 0, m)),
        ],
        out_specs=[
            pl.BlockSpec((1, 1, N), lambda b, m: (b, 0, 0)),
            pl.BlockSpec((1, 1, _TM), lambda b, m: (b, 0, m)),
        ],
        out_shape=[
            jax.ShapeDtypeStruct((B, 1, N), jnp.float32),
            jax.ShapeDtypeStruct((B, 1, M), jnp.float32),
        ],
    )(x, y)
    return jnp.mean(dx) + jnp.mean(dy) * 10.0
